# baseline (speedup 1.0000x reference)
import jax
import jax.numpy as jnp
from jax.experimental import pallas as pl
from jax.experimental.pallas import tpu as pltpu

B = 2048
L = 32
D_MODEL = 128
D_LLM = 1024
D_FF = 512
E = 8
D_IN = L * D_MODEL
EPS = 1e-09

BB = 256
N_EG = E * D_MODEL + D_MODEL
N_ALL = D_FF + N_EG
RPREP = 512


def _prep_kernel(Wgf_ref, We_ref, Wgen_ref, Wgd_ref, Wgo_ref,
                 Wall_ref, Wgdb_ref, Wgob_ref):
    Wall_ref[:, 0:D_FF] = Wgf_ref[...].astype(jnp.bfloat16)
    for e in range(E):
        Wall_ref[:, D_FF + e * D_MODEL:D_FF + (e + 1) * D_MODEL] = (
            We_ref[e].astype(jnp.bfloat16))
    Wall_ref[:, D_FF + E * D_MODEL:] = Wgen_ref[...].astype(jnp.bfloat16)
    Wgdb_ref[...] = Wgd_ref[...].astype(jnp.bfloat16)
    Wgob_ref[...] = Wgo_ref[...].astype(jnp.bfloat16)


def _moe_kernel(x_ref, dkp_ref, cyc_ref, Wall_ref, Wgdb_ref, Wgc_ref, bg_ref,
                Wgob_ref, bgo_ref, be_ref, bgen_ref, out_ref):
    xb = x_ref[...].reshape(BB, D_IN).astype(jnp.bfloat16)
    dkpb = dkp_ref[...].astype(jnp.bfloat16)

    zg = jnp.dot(xb, Wall_ref[:, 0:D_FF], preferred_element_type=jnp.float32)
    zd = jnp.dot(dkpb, Wgdb_ref[...], preferred_element_type=jnp.float32)
    big = jnp.dot(xb, Wall_ref[:, D_FF:], preferred_element_type=jnp.float32)

    z = zg + zd + cyc_ref[...] * Wgc_ref[...] + bg_ref[...]
    hb = jax.nn.gelu(z).astype(jnp.bfloat16)
    logits = jnp.dot(hb, Wgob_ref[...], preferred_element_type=jnp.float32) + bgo_ref[...]

    cols = jax.lax.broadcasted_iota(jnp.int32, (BB, E), 1)
    m1 = jnp.max(logits, axis=1, keepdims=True)
    a1 = jnp.min(jnp.where(logits == m1, cols, E), axis=1, keepdims=True)
    sel1 = cols == a1
    rest = jnp.where(sel1, -jnp.inf, logits)
    m2 = jnp.max(rest, axis=1, keepdims=True)
    a2 = jnp.min(jnp.where(rest == m2, cols, E), axis=1, keepdims=True)
    sel = sel1 | (cols == a2)

    p = jnp.exp(logits - m1)
    probs = p / jnp.sum(p, axis=1, keepdims=True)
    gated = jnp.where(sel, probs, 0.0)
    gates = gated / (jnp.sum(gated, axis=1, keepdims=True) + EPS)
    gates_b = gates.astype(jnp.bfloat16).astype(jnp.float32)

    terms = []
    for e in range(E):
        pe = big[:, e * D_MODEL:(e + 1) * D_MODEL] + be_ref[e]
        pe_b = pe.astype(jnp.bfloat16).astype(jnp.float32)
        terms.append(gates_b[:, e:e + 1] * pe_b)
    while len(terms) > 1:
        terms = [terms[i] + terms[i + 1] for i in range(0, len(terms), 2)]
    combined = terms[0].astype(jnp.bfloat16).astype(jnp.float32)

    gen = big[:, E * D_MODEL:]
    out_ref[...] = gen + bgen_ref[...] + combined


def kernel(cycle_curve_data, cycle_numbers, DKP_embeddings, Wg_dkp, Wg_cyc,
           Wg_flat, bg, Wg_out, bg_out, We, be, Wgen, bgen):
    b = cycle_curve_data.shape[0]
    bg2 = bg.reshape(1, -1)
    bgo2 = bg_out.reshape(1, -1)
    bgen2 = bgen.reshape(1, -1)

    pgrid = D_IN // RPREP
    Wall, Wgdb, Wgob = pl.pallas_call(
        _prep_kernel,
        grid=(pgrid,),
        in_specs=[
            pl.BlockSpec((RPREP, D_FF), lambda i: (i, 0)),
            pl.BlockSpec((E, RPREP, D_MODEL), lambda i: (0, i, 0)),
            pl.BlockSpec((RPREP, D_MODEL), lambda i: (i, 0)),
            pl.BlockSpec((D_LLM // pgrid, D_FF), lambda i: (i, 0)),
            pl.BlockSpec((D_FF // pgrid, E), lambda i: (i, 0)),
        ],
        out_specs=[
            pl.BlockSpec((RPREP, N_ALL), lambda i: (i, 0)),
            pl.BlockSpec((D_LLM // pgrid, D_FF), lambda i: (i, 0)),
            pl.BlockSpec((D_FF // pgrid, E), lambda i: (i, 0)),
        ],
        out_shape=[
            jax.ShapeDtypeStruct((D_IN, N_ALL), jnp.bfloat16),
            jax.ShapeDtypeStruct((D_LLM, D_FF), jnp.bfloat16),
            jax.ShapeDtypeStruct((D_FF, E), jnp.bfloat16),
        ],
        compiler_params=pltpu.CompilerParams(
            dimension_semantics=("arbitrary",),
        ),
    )(Wg_flat, We, Wgen, Wg_dkp, Wg_out)

    grid = b // BB
    out = pl.pallas_call(
        _moe_kernel,
        grid=(grid,),
        in_specs=[
            pl.BlockSpec((BB, L, D_MODEL), lambda i: (i, 0, 0)),
            pl.BlockSpec((BB, D_LLM), lambda i: (i, 0)),
            pl.BlockSpec((BB, 1), lambda i: (i, 0)),
            pl.BlockSpec((D_IN, N_ALL), lambda i: (0, 0)),
            pl.BlockSpec((D_LLM, D_FF), lambda i: (0, 0)),
            pl.BlockSpec((1, D_FF), lambda i: (0, 0)),
            pl.BlockSpec((1, D_FF), lambda i: (0, 0)),
            pl.BlockSpec((D_FF, E), lambda i: (0, 0)),
            pl.BlockSpec((1, E), lambda i: (0, 0)),
            pl.BlockSpec((E, D_MODEL), lambda i: (0, 0)),
            pl.BlockSpec((1, D_MODEL), lambda i: (0, 0)),
        ],
        out_specs=pl.BlockSpec((BB, D_MODEL), lambda i: (i, 0)),
        out_shape=jax.ShapeDtypeStruct((b, D_MODEL), jnp.float32),
        compiler_params=pltpu.CompilerParams(
            dimension_semantics=("arbitrary",),
        ),
    )(cycle_curve_data, DKP_embeddings, cycle_numbers, Wall, Wgdb, Wg_cyc,
      bg2, Wgob, bgo2, be, bgen2)
    return (out, jnp.float32(0.0))
